# baseline (speedup 1.0000x reference)
import jax
import jax.numpy as jnp
from jax import lax
from jax.experimental import pallas as pl
from jax.experimental.pallas import tpu as pltpu
from jax.experimental.pallas import tpu_sc as plsc

_W = 224
_ROWS = 16 * 96 * 224

_NC = 2
_NS = 16
_NW = _NC * _NS
_SC_ROWS = 32768
_RW = _SC_ROWS // _NW
_SC_CH = 128
_NCH = _RW // _SC_CH
_SC_BASE = _ROWS - _SC_ROWS

_B = 8192
_GRID = _SC_BASE // _B
_CH = 64


def _tc_loss_kernel(out_ref, tgt_ref, res_ref):
    acc = None
    for r in range(0, _B, _CH):
        o = out_ref[pl.ds(r, _CH), :]
        t = tgt_ref[pl.ds(r, _CH), :]
        d = o - t
        a = jnp.abs(d)
        m = jnp.minimum(a, 1.0)
        f = m * (a - 0.5 * m)
        f = jnp.where(t != 0.0, f, 0.0)
        p = jnp.sum(f.reshape(-1, 8, _W), axis=0)
        acc = p if acc is None else acc + p
    res_ref[...] = jnp.sum(acc)[None, None, None]


def _sc_loss_kernel(out_hbm, tgt_hbm, res_hbm, obuf, tbuf, acc_ref):
    c = lax.axis_index("c")
    s = lax.axis_index("s")
    wid = s * _NC + c
    base = _SC_BASE + wid * _RW
    acc_ref[...] = jnp.zeros((16,), jnp.float32)

    def chunk_body(k, carry):
        r0 = base + k * _SC_CH
        pltpu.sync_copy(out_hbm.at[pl.ds(r0, _SC_CH), :], obuf)
        pltpu.sync_copy(tgt_hbm.at[pl.ds(r0, _SC_CH), :], tbuf)

        def row_body(r, c2):
            for cc in range(_W // 16):
                o = obuf[r, pl.ds(cc * 16, 16)]
                t = tbuf[r, pl.ds(cc * 16, 16)]
                d = o - t
                a = jnp.abs(d)
                m = jnp.minimum(a, 1.0)
                f = m * (a - 0.5 * m)
                f = jnp.where(t != 0.0, f, 0.0)
                acc_ref[...] = acc_ref[...] + f
            return c2

        return lax.fori_loop(0, _SC_CH, row_body, carry)

    lax.fori_loop(0, _NCH, chunk_body, 0)
    pltpu.sync_copy(acc_ref, res_hbm.at[wid])


_sc_call = pl.kernel(
    _sc_loss_kernel,
    out_type=jax.ShapeDtypeStruct((_NW, 16), jnp.float32),
    mesh=plsc.VectorSubcoreMesh(core_axis_name="c", subcore_axis_name="s"),
    scratch_types=[
        pltpu.VMEM((_SC_CH, _W), jnp.float32),
        pltpu.VMEM((_SC_CH, _W), jnp.float32),
        pltpu.VMEM((16,), jnp.float32),
    ],
)


def kernel(out, target):
    o2 = out.reshape(_ROWS, _W)
    t2 = target.reshape(_ROWS, _W)
    res_tc = pl.pallas_call(
        _tc_loss_kernel,
        grid=(_GRID,),
        in_specs=[
            pl.BlockSpec((_B, _W), lambda i: (i, 0)),
            pl.BlockSpec((_B, _W), lambda i: (i, 0)),
        ],
        out_specs=pl.BlockSpec((1, 1, 1), lambda i: (i, 0, 0)),
        out_shape=jax.ShapeDtypeStruct((_GRID, 1, 1), jnp.float32),
        compiler_params=pltpu.CompilerParams(
            dimension_semantics=("parallel",),
        ),
    )(o2, t2)
    res_sc = _sc_call(o2, t2)
    return jnp.sum(res_tc) + jnp.sum(res_sc)


# speedup vs baseline: 1.0184x; 1.0184x over previous
import jax
import jax.numpy as jnp
from jax import lax
from jax.experimental import pallas as pl
from jax.experimental.pallas import tpu as pltpu
from jax.experimental.pallas import tpu_sc as plsc

_W = 224
_ROWS = 16 * 96 * 224

_NC = 2
_NS = 16
_NW = _NC * _NS
_SC_ROWS = 32768
_RW = _SC_ROWS // _NW
_SC_CH = 64
_NCH = _RW // _SC_CH
_SC_BASE = _ROWS - _SC_ROWS

_B = 8192
_GRID = _SC_BASE // _B
_CH = 64


def _tc_loss_kernel(out_ref, tgt_ref, res_ref):
    acc = None
    for r in range(0, _B, _CH):
        o = out_ref[pl.ds(r, _CH), :]
        t = tgt_ref[pl.ds(r, _CH), :]
        d = o - t
        a = jnp.abs(d)
        m = jnp.minimum(a, 1.0)
        f = m * (a - 0.5 * m)
        f = jnp.where(t != 0.0, f, 0.0)
        p = jnp.sum(f.reshape(-1, 8, _W), axis=0)
        acc = p if acc is None else acc + p
    res_ref[...] = jnp.sum(acc)[None, None, None]


def _sc_chunk_sum(obuf, tbuf, acc):
    def row_body(r, a_c):
        for cc in range(_W // 16):
            o = obuf[r, pl.ds(cc * 16, 16)]
            t = tbuf[r, pl.ds(cc * 16, 16)]
            d = o - t
            a = jnp.abs(d)
            m = jnp.minimum(a, 1.0)
            f = m * (a - 0.5 * m)
            f = jnp.where(t != 0.0, f, 0.0)
            a_c = a_c + f
        return a_c

    return lax.fori_loop(0, _SC_CH, row_body, acc)


def _sc_loss_kernel(out_hbm, tgt_hbm, res_hbm,
                    obuf0, tbuf0, obuf1, tbuf1, acc_ref,
                    osem0, tsem0, osem1, tsem1):
    c = lax.axis_index("c")
    s = lax.axis_index("s")
    wid = s * _NC + c
    base = _SC_BASE + wid * _RW

    def start(k, obuf, tbuf, osem, tsem):
        r0 = base + k * _SC_CH
        pltpu.make_async_copy(out_hbm.at[pl.ds(r0, _SC_CH), :], obuf, osem).start()
        pltpu.make_async_copy(tgt_hbm.at[pl.ds(r0, _SC_CH), :], tbuf, tsem).start()

    def wait(k, obuf, tbuf, osem, tsem):
        r0 = base + k * _SC_CH
        pltpu.make_async_copy(out_hbm.at[pl.ds(r0, _SC_CH), :], obuf, osem).wait()
        pltpu.make_async_copy(tgt_hbm.at[pl.ds(r0, _SC_CH), :], tbuf, tsem).wait()

    start(0, obuf0, tbuf0, osem0, tsem0)

    def pair_body(kk, acc):
        c0 = 2 * kk
        c1 = 2 * kk + 1
        start(c1, obuf1, tbuf1, osem1, tsem1)
        wait(c0, obuf0, tbuf0, osem0, tsem0)
        acc = _sc_chunk_sum(obuf0, tbuf0, acc)

        @pl.when(c1 + 1 < _NCH)
        def _():
            start(c1 + 1, obuf0, tbuf0, osem0, tsem0)

        wait(c1, obuf1, tbuf1, osem1, tsem1)
        return _sc_chunk_sum(obuf1, tbuf1, acc)

    acc = lax.fori_loop(0, _NCH // 2, pair_body, jnp.zeros((16,), jnp.float32))
    acc_ref[...] = acc
    pltpu.sync_copy(acc_ref, res_hbm.at[wid])


_sc_call = pl.kernel(
    _sc_loss_kernel,
    out_type=jax.ShapeDtypeStruct((_NW, 16), jnp.float32),
    mesh=plsc.VectorSubcoreMesh(core_axis_name="c", subcore_axis_name="s"),
    scratch_types=[
        pltpu.VMEM((_SC_CH, _W), jnp.float32),
        pltpu.VMEM((_SC_CH, _W), jnp.float32),
        pltpu.VMEM((_SC_CH, _W), jnp.float32),
        pltpu.VMEM((_SC_CH, _W), jnp.float32),
        pltpu.VMEM((16,), jnp.float32),
        pltpu.SemaphoreType.DMA,
        pltpu.SemaphoreType.DMA,
        pltpu.SemaphoreType.DMA,
        pltpu.SemaphoreType.DMA,
    ],
)


def kernel(out, target):
    o2 = out.reshape(_ROWS, _W)
    t2 = target.reshape(_ROWS, _W)
    res_tc = pl.pallas_call(
        _tc_loss_kernel,
        grid=(_GRID,),
        in_specs=[
            pl.BlockSpec((_B, _W), lambda i: (i, 0)),
            pl.BlockSpec((_B, _W), lambda i: (i, 0)),
        ],
        out_specs=pl.BlockSpec((1, 1, 1), lambda i: (i, 0, 0)),
        out_shape=jax.ShapeDtypeStruct((_GRID, 1, 1), jnp.float32),
        compiler_params=pltpu.CompilerParams(
            dimension_semantics=("parallel",),
        ),
    )(o2, t2)
    res_sc = _sc_call(o2, t2)
    return jnp.sum(res_tc) + jnp.sum(res_sc)


# speedup vs baseline: 1.0225x; 1.0040x over previous
import jax
import jax.numpy as jnp
from jax import lax
from jax.experimental import pallas as pl
from jax.experimental.pallas import tpu as pltpu
from jax.experimental.pallas import tpu_sc as plsc

_W = 224
_ROWS = 16 * 96 * 224

_NC = 2
_NS = 16
_NW = _NC * _NS
_SC_ROWS = 16384
_RW = _SC_ROWS // _NW
_SC_CH = 64
_NCH = _RW // _SC_CH
_SC_BASE = _ROWS - _SC_ROWS

_B = 8192
_GRID = _SC_BASE // _B
_CH = 64


def _tc_loss_kernel(out_ref, tgt_ref, res_ref):
    acc = None
    for r in range(0, _B, _CH):
        o = out_ref[pl.ds(r, _CH), :]
        t = tgt_ref[pl.ds(r, _CH), :]
        d = o - t
        a = jnp.abs(d)
        m = jnp.minimum(a, 1.0)
        f = m * (a - 0.5 * m)
        f = jnp.where(t != 0.0, f, 0.0)
        p = jnp.sum(f.reshape(-1, 8, _W), axis=0)
        acc = p if acc is None else acc + p
    res_ref[...] = jnp.sum(acc)[None, None, None]


def _sc_chunk_sum(obuf, tbuf, acc):
    def row_body(r, a_c):
        for cc in range(_W // 16):
            o = obuf[r, pl.ds(cc * 16, 16)]
            t = tbuf[r, pl.ds(cc * 16, 16)]
            d = o - t
            a = jnp.abs(d)
            m = jnp.minimum(a, 1.0)
            f = m * (a - 0.5 * m)
            f = jnp.where(t != 0.0, f, 0.0)
            a_c = a_c + f
        return a_c

    return lax.fori_loop(0, _SC_CH, row_body, acc)


def _sc_loss_kernel(out_hbm, tgt_hbm, res_hbm,
                    obuf0, tbuf0, obuf1, tbuf1, acc_ref,
                    osem0, tsem0, osem1, tsem1):
    c = lax.axis_index("c")
    s = lax.axis_index("s")
    wid = s * _NC + c
    base = _SC_BASE + wid * _RW

    def start(k, obuf, tbuf, osem, tsem):
        r0 = base + k * _SC_CH
        pltpu.make_async_copy(out_hbm.at[pl.ds(r0, _SC_CH), :], obuf, osem).start()
        pltpu.make_async_copy(tgt_hbm.at[pl.ds(r0, _SC_CH), :], tbuf, tsem).start()

    def wait(k, obuf, tbuf, osem, tsem):
        r0 = base + k * _SC_CH
        pltpu.make_async_copy(out_hbm.at[pl.ds(r0, _SC_CH), :], obuf, osem).wait()
        pltpu.make_async_copy(tgt_hbm.at[pl.ds(r0, _SC_CH), :], tbuf, tsem).wait()

    start(0, obuf0, tbuf0, osem0, tsem0)

    def pair_body(kk, acc):
        c0 = 2 * kk
        c1 = 2 * kk + 1
        start(c1, obuf1, tbuf1, osem1, tsem1)
        wait(c0, obuf0, tbuf0, osem0, tsem0)
        acc = _sc_chunk_sum(obuf0, tbuf0, acc)

        @pl.when(c1 + 1 < _NCH)
        def _():
            start(c1 + 1, obuf0, tbuf0, osem0, tsem0)

        wait(c1, obuf1, tbuf1, osem1, tsem1)
        return _sc_chunk_sum(obuf1, tbuf1, acc)

    acc = lax.fori_loop(0, _NCH // 2, pair_body, jnp.zeros((16,), jnp.float32))
    acc_ref[...] = acc
    pltpu.sync_copy(acc_ref, res_hbm.at[wid])


_sc_call = pl.kernel(
    _sc_loss_kernel,
    out_type=jax.ShapeDtypeStruct((_NW, 16), jnp.float32),
    mesh=plsc.VectorSubcoreMesh(core_axis_name="c", subcore_axis_name="s"),
    scratch_types=[
        pltpu.VMEM((_SC_CH, _W), jnp.float32),
        pltpu.VMEM((_SC_CH, _W), jnp.float32),
        pltpu.VMEM((_SC_CH, _W), jnp.float32),
        pltpu.VMEM((_SC_CH, _W), jnp.float32),
        pltpu.VMEM((16,), jnp.float32),
        pltpu.SemaphoreType.DMA,
        pltpu.SemaphoreType.DMA,
        pltpu.SemaphoreType.DMA,
        pltpu.SemaphoreType.DMA,
    ],
)


def kernel(out, target):
    o2 = out.reshape(_ROWS, _W)
    t2 = target.reshape(_ROWS, _W)
    res_tc = pl.pallas_call(
        _tc_loss_kernel,
        grid=(_GRID,),
        in_specs=[
            pl.BlockSpec((_B, _W), lambda i: (i, 0)),
            pl.BlockSpec((_B, _W), lambda i: (i, 0)),
        ],
        out_specs=pl.BlockSpec((1, 1, 1), lambda i: (i, 0, 0)),
        out_shape=jax.ShapeDtypeStruct((_GRID, 1, 1), jnp.float32),
        compiler_params=pltpu.CompilerParams(
            dimension_semantics=("parallel",),
        ),
    )(o2, t2)
    res_sc = _sc_call(o2, t2)
    return jnp.sum(res_tc) + jnp.sum(res_sc)


# speedup vs baseline: 1.1163x; 1.0917x over previous
import jax
import jax.numpy as jnp
from jax.experimental import pallas as pl
from jax.experimental.pallas import tpu as pltpu

_W = 224
_ROWS = 16 * 96 * 224
_B = 14336
_GRID = _ROWS // _B


_CH = 64


def _loss_kernel(out_ref, tgt_ref, res_ref):
    acc = None
    for r in range(0, _B, _CH):
        o = out_ref[pl.ds(r, _CH), :]
        t = tgt_ref[pl.ds(r, _CH), :]
        d = o - t
        a = jnp.abs(d)
        m = jnp.minimum(a, 1.0)
        f = m * (a - 0.5 * m)
        f = jnp.where(t != 0.0, f, 0.0)
        p = jnp.sum(f.reshape(-1, 8, _W), axis=0)
        acc = p if acc is None else acc + p
    res_ref[...] = jnp.sum(acc)[None, None, None]


def kernel(out, target):
    o2 = out.reshape(_ROWS, _W)
    t2 = target.reshape(_ROWS, _W)
    res = pl.pallas_call(
        _loss_kernel,
        grid=(_GRID,),
        in_specs=[
            pl.BlockSpec((_B, _W), lambda i: (i, 0)),
            pl.BlockSpec((_B, _W), lambda i: (i, 0)),
        ],
        out_specs=pl.BlockSpec((1, 1, 1), lambda i: (i, 0, 0)),
        out_shape=jax.ShapeDtypeStruct((_GRID, 1, 1), jnp.float32),
        compiler_params=pltpu.CompilerParams(
            dimension_semantics=("parallel",),
            vmem_limit_bytes=100 * 1024 * 1024,
        ),
    )(o2, t2)
    return jnp.sum(res)


# speedup vs baseline: 1.1262x; 1.0089x over previous
import jax
import jax.numpy as jnp
from jax.experimental import pallas as pl
from jax.experimental.pallas import tpu as pltpu

_W = 224
_ROWS = 16 * 96 * 224
_B = 14336
_GRID = _ROWS // _B
_CH = 64


def _loss_kernel(out_ref, tgt_ref, res_ref, acc_ref):
    i = pl.program_id(0)
    acc = None
    for r in range(0, _B, _CH):
        o = out_ref[pl.ds(r, _CH), :]
        t = tgt_ref[pl.ds(r, _CH), :]
        d = o - t
        a = jnp.abs(d)
        m = jnp.minimum(a, 1.0)
        f = m * (a - 0.5 * m)
        f = jnp.where(t != 0.0, f, 0.0)
        p = jnp.sum(f.reshape(-1, 8, _W), axis=0)
        acc = p if acc is None else acc + p

    @pl.when(i == 0)
    def _init():
        acc_ref[...] = acc

    @pl.when(i > 0)
    def _acc():
        acc_ref[...] = acc_ref[...] + acc

    @pl.when(i == _GRID - 1)
    def _fin():
        res_ref[...] = jnp.sum(acc_ref[...])[None, None]


def kernel(out, target):
    o2 = out.reshape(_ROWS, _W)
    t2 = target.reshape(_ROWS, _W)
    res = pl.pallas_call(
        _loss_kernel,
        grid=(_GRID,),
        in_specs=[
            pl.BlockSpec((_B, _W), lambda i: (i, 0)),
            pl.BlockSpec((_B, _W), lambda i: (i, 0)),
        ],
        out_specs=pl.BlockSpec((1, 1), lambda i: (0, 0)),
        out_shape=jax.ShapeDtypeStruct((1, 1), jnp.float32),
        scratch_shapes=[pltpu.VMEM((8, _W), jnp.float32)],
        compiler_params=pltpu.CompilerParams(
            dimension_semantics=("arbitrary",),
            vmem_limit_bytes=100 * 1024 * 1024,
        ),
    )(o2, t2)
    return res[0, 0]
